# baseline (speedup 1.0000x reference)
import functools

import jax
import jax.numpy as jnp
from jax import lax
from jax.experimental import pallas as pl
from jax.experimental.pallas import tpu as pltpu
from jax.experimental.pallas import tpu_sc as plsc

NC = 2
NS = 16
L = 16
K = 128
FH = 128
F = 2 * FH
GRAPH = 1000
NB = 2


def _sc_mesh():
    return plsc.VectorSubcoreMesh(
        core_axis_name="c", subcore_axis_name="s", num_cores=NC, num_subcores=NS
    )


@functools.lru_cache(maxsize=None)
def _degree_call(e_pad: int, n_pad: int):
    per_tile = e_pad // NS
    npad_per_tile = n_pad // NS

    @functools.partial(
        pl.kernel,
        out_type=jax.ShapeDtypeStruct((NC * n_pad,), jnp.float32),
        mesh=_sc_mesh(),
        compiler_params=pltpu.CompilerParams(needs_layout_passes=False),
        scratch_types=[
            pltpu.VMEM((per_tile,), jnp.int32),
            pltpu.VMEM((n_pad,), jnp.float32),
            pltpu.VMEM((NS, npad_per_tile), jnp.float32),
            pltpu.VMEM((npad_per_tile,), jnp.float32),
            pltpu.VMEM_SHARED((NS, n_pad), jnp.float32),
        ],
    )
    def deg_kernel(idx_hbm, deg_hbm, idx_v, hist_v, part_v, red_v, part_sh):
        c = lax.axis_index("c")
        s = lax.axis_index("s")

        def zbody(i, _):
            hist_v[pl.ds(i * L, L)] = jnp.zeros((L,), jnp.float32)
            return 0

        lax.fori_loop(0, n_pad // L, zbody, 0)

        pltpu.sync_copy(idx_hbm.at[pl.ds(c * e_pad + s * per_tile, per_tile)], idx_v)
        ones = jnp.ones((L,), jnp.float32)

        def hbody(i, _):
            vec = idx_v[pl.ds(i * L, L)]
            plsc.addupdate_scatter(hist_v, [vec], ones)
            return 0

        lax.fori_loop(0, per_tile // L, hbody, 0)

        pltpu.sync_copy(hist_v, part_sh.at[s])
        plsc.subcore_barrier()
        pltpu.sync_copy(
            part_sh.at[:, pl.ds(s * npad_per_tile, npad_per_tile)], part_v
        )

        def rbody(i, _):
            acc = part_v[0, pl.ds(i * L, L)]
            for p in range(1, NS):
                acc = acc + part_v[p, pl.ds(i * L, L)]
            red_v[pl.ds(i * L, L)] = acc
            return 0

        lax.fori_loop(0, npad_per_tile // L, rbody, 0)
        pltpu.sync_copy(
            red_v, deg_hbm.at[pl.ds(c * n_pad + s * npad_per_tile, npad_per_tile)]
        )

    return deg_kernel


@functools.lru_cache(maxsize=None)
def _agg_call(e_rows: int, n_pad: int):
    cpt = e_rows // NS
    npad_per_tile = n_pad // NS

    @functools.partial(
        pl.kernel,
        out_type=jax.ShapeDtypeStruct((NC, n_pad, FH), jnp.float32),
        mesh=_sc_mesh(),
        compiler_params=pltpu.CompilerParams(needs_layout_passes=False),
        scratch_types=[
            pltpu.VMEM((cpt // 2, K), jnp.int32),
            pltpu.VMEM((cpt // 2, K), jnp.int32),
            pltpu.VMEM((NB, K, FH), jnp.float32),
            pltpu.VMEM_SHARED((n_pad, FH), jnp.float32),
            [pltpu.SemaphoreType.DMA] * NB,
            [pltpu.SemaphoreType.DMA] * NB,
        ],
    )
    def agg_kernel(h_hbm, src_hbm, dst_hbm, out_hbm, src_v, dst_v, rows_v, acc_sh, gs, ss):
        c = lax.axis_index("c")
        s = lax.axis_index("s")
        hb = cpt // 2

        def zbody(i, _):
            r = i // (FH // L)
            j = i % (FH // L)
            rows_v[0, r, pl.ds(j * L, L)] = jnp.zeros((L,), jnp.float32)
            return 0

        lax.fori_loop(0, K * FH // L, zbody, 0)
        for t in range(npad_per_tile // K):
            pltpu.sync_copy(
                rows_v.at[0], acc_sh.at[pl.ds(s * npad_per_tile + t * K, K)]
            )
        plsc.subcore_barrier()

        for h in range(2):
            pltpu.sync_copy(
                src_hbm.at[pl.ds(c * e_rows + s * cpt + h * hb, hb)], src_v
            )
            pltpu.sync_copy(dst_hbm.at[pl.ds(s * cpt + h * hb, hb)], dst_v)
            for b in range(NB - 1):
                pltpu.async_copy(h_hbm.at[src_v.at[b]], rows_v.at[b], gs[b])

            @pl.loop(0, hb, step=NB)
            def ebody(t):
                for b in range(NB):
                    chunk = t + b
                    nxt = chunk + NB - 1
                    nb = (b + NB - 1) % NB

                    pltpu.make_async_copy(
                        h_hbm.at[src_v.at[chunk]], rows_v.at[b], gs[b]
                    ).wait()
                    pltpu.async_copy(
                        rows_v.at[b], acc_sh.at[dst_v.at[chunk]], ss[b], add=True
                    )

                    @pl.when(nxt < hb)
                    def _():
                        @pl.when(chunk >= 1)
                        def _():
                            pltpu.make_async_copy(
                                rows_v.at[nb],
                                acc_sh.at[dst_v.at[chunk - 1]],
                                ss[nb],
                            ).wait()

                        pltpu.async_copy(
                            h_hbm.at[src_v.at[nxt]], rows_v.at[nb], gs[nb]
                        )

            for d in range(NB):
                cd = hb - NB + d
                pltpu.make_async_copy(
                    rows_v.at[cd % NB], acc_sh.at[dst_v.at[cd]], ss[cd % NB]
                ).wait()

        plsc.subcore_barrier()
        pltpu.sync_copy(
            acc_sh.at[pl.ds(s * npad_per_tile, npad_per_tile)],
            out_hbm.at[c, pl.ds(s * npad_per_tile, npad_per_tile)],
        )

    return agg_kernel


def _scale_call(x_pad, deg_cols, n_pad):
    rb = 1024

    def body(x_ref, deg_ref, h_ref, norms_ref):
        ns = lax.rsqrt(jnp.clip(deg_ref[:, 0:1], 1.0, None))
        nd = lax.rsqrt(jnp.clip(deg_ref[:, 1:2], 1.0, None))
        norms_ref[:, 0:1] = ns
        norms_ref[:, 1:2] = nd
        h = x_ref[:] * ns
        h_ref[0] = h[:, :FH]
        h_ref[1] = h[:, FH:]

    return pl.pallas_call(
        body,
        grid=(n_pad // rb,),
        in_specs=[
            pl.BlockSpec((rb, F), lambda i: (i, 0)),
            pl.BlockSpec((rb, 2), lambda i: (i, 0)),
        ],
        out_specs=[
            pl.BlockSpec((NC, rb, FH), lambda i: (0, i, 0)),
            pl.BlockSpec((rb, 2), lambda i: (i, 0)),
        ],
        out_shape=[
            jax.ShapeDtypeStruct((NC, n_pad, FH), jnp.float32),
            jax.ShapeDtypeStruct((n_pad, 2), jnp.float32),
        ],
    )(x_pad, deg_cols)


def _mid_call(agg1, norms, W1, b1, W2, n_pad):
    rb = 512
    h1f = W1.shape[1]

    def body(agg_ref, norms_ref, w1_ref, b1_ref, w2_ref, out_ref):
        a = jnp.concatenate([agg_ref[0], agg_ref[1]], axis=1)
        h = a * norms_ref[:, 1:2]
        h1 = jnp.dot(h, w1_ref[:], preferred_element_type=jnp.float32) + b1_ref[:]
        h1 = jnp.where(h1 > 0, h1, 0.01 * h1)
        m = jnp.dot(h1, w2_ref[:], preferred_element_type=jnp.float32)
        ms = m * norms_ref[:, 0:1]
        out_ref[0] = ms[:, :FH]
        out_ref[1] = ms[:, FH:]

    return pl.pallas_call(
        body,
        grid=(n_pad // rb,),
        in_specs=[
            pl.BlockSpec((NC, rb, FH), lambda i: (0, i, 0)),
            pl.BlockSpec((rb, 2), lambda i: (i, 0)),
            pl.BlockSpec((F, h1f), lambda i: (0, 0)),
            pl.BlockSpec((1, h1f), lambda i: (0, 0)),
            pl.BlockSpec((h1f, F), lambda i: (0, 0)),
        ],
        out_specs=pl.BlockSpec((NC, rb, FH), lambda i: (0, i, 0)),
        out_shape=jax.ShapeDtypeStruct((NC, n_pad, FH), jnp.float32),
    )(agg1, norms, W1, b1, W2)


def _readout_call(agg2, norms, b2, batch):
    def body(agg_ref, norms_ref, b2_ref, out_ref):
        a = jnp.concatenate([agg_ref[0], agg_ref[1]], axis=1)
        h2 = a * norms_ref[:, 1:2] + b2_ref[:]
        h2 = jnp.where(h2 > 0, h2, 0.01 * h2)
        out_ref[0] = jnp.mean(h2, axis=0, keepdims=True)

    out = pl.pallas_call(
        body,
        grid=(batch,),
        in_specs=[
            pl.BlockSpec((NC, GRAPH, FH), lambda i: (0, i, 0)),
            pl.BlockSpec((GRAPH, 2), lambda i: (i, 0)),
            pl.BlockSpec((1, F), lambda i: (0, 0)),
        ],
        out_specs=pl.BlockSpec((1, 1, F), lambda i: (i, 0, 0)),
        out_shape=jax.ShapeDtypeStruct((batch, 1, F), jnp.float32),
    )(agg2, norms, b2)
    return out.reshape(batch, F)


def kernel(x, edge_index, W1, b1, W2, b2):
    n = x.shape[0]
    batch = n // GRAPH
    n_pad = ((n + 1 + 1023) // 1024) * 1024

    src = edge_index[0].astype(jnp.int32)
    dst = edge_index[1].astype(jnp.int32)
    e = src.shape[0]
    ec = NS * K * 8
    e_pad = ((e + ec - 1) // ec) * ec
    e_rows = e_pad // K
    src_p = jnp.full((e_pad,), n, jnp.int32).at[:e].set(src)
    dst_p = jnp.full((e_pad,), n, jnp.int32).at[:e].set(dst)
    idx2 = jnp.concatenate([src_p, dst_p])
    src2 = jnp.concatenate([src_p, src_p + n_pad]).reshape(NC * e_rows, K)
    dst2 = dst_p.reshape(e_rows, K)

    x_pad = jnp.zeros((n_pad, F), jnp.float32).at[:n].set(x)

    deg = _degree_call(e_pad, n_pad)(idx2)
    h_split, norms = _scale_call(x_pad, deg.reshape(NC, n_pad).T, n_pad)
    agg1 = _agg_call(e_rows, n_pad)(
        h_split.reshape(NC * n_pad, FH), src2, dst2
    )
    m_split = _mid_call(agg1, norms, W1, b1.reshape(1, -1), W2, n_pad)
    agg2 = _agg_call(e_rows, n_pad)(
        m_split.reshape(NC * n_pad, FH), src2, dst2
    )
    return _readout_call(agg2, norms, b2.reshape(1, -1), batch)


# speedup vs baseline: 1.0932x; 1.0932x over previous
import functools

import jax
import jax.numpy as jnp
from jax import lax
from jax.experimental import pallas as pl
from jax.experimental.pallas import tpu as pltpu
from jax.experimental.pallas import tpu_sc as plsc

NC = 2
NS = 16
L = 16
K = 128
FH = 128
F = 2 * FH
GRAPH = 1000
NB = 4
CW = 64
WN = 4


def _sc_mesh():
    return plsc.VectorSubcoreMesh(
        core_axis_name="c", subcore_axis_name="s", num_cores=NC, num_subcores=NS
    )


@functools.lru_cache(maxsize=None)
def _degree_call(e_pad: int, n_pad: int):
    per_tile = e_pad // NS
    npad_per_tile = n_pad // NS

    @functools.partial(
        pl.kernel,
        out_type=jax.ShapeDtypeStruct((NC * n_pad,), jnp.float32),
        mesh=_sc_mesh(),
        compiler_params=pltpu.CompilerParams(needs_layout_passes=False),
        scratch_types=[
            pltpu.VMEM((per_tile,), jnp.int32),
            pltpu.VMEM((n_pad,), jnp.float32),
            pltpu.VMEM((NS, npad_per_tile), jnp.float32),
            pltpu.VMEM((npad_per_tile,), jnp.float32),
            pltpu.VMEM_SHARED((NS, n_pad), jnp.float32),
        ],
    )
    def deg_kernel(idx_hbm, deg_hbm, idx_v, hist_v, part_v, red_v, part_sh):
        c = lax.axis_index("c")
        s = lax.axis_index("s")

        def zbody(i, _):
            hist_v[pl.ds(i * L, L)] = jnp.zeros((L,), jnp.float32)
            return 0

        lax.fori_loop(0, n_pad // L, zbody, 0)

        pltpu.sync_copy(idx_hbm.at[pl.ds(c * e_pad + s * per_tile, per_tile)], idx_v)
        ones = jnp.ones((L,), jnp.float32)

        def hbody(i, _):
            vec = idx_v[pl.ds(i * L, L)]
            plsc.addupdate_scatter(hist_v, [vec], ones)
            return 0

        lax.fori_loop(0, per_tile // L, hbody, 0)

        pltpu.sync_copy(hist_v, part_sh.at[s])
        plsc.subcore_barrier()
        pltpu.sync_copy(
            part_sh.at[:, pl.ds(s * npad_per_tile, npad_per_tile)], part_v
        )

        def rbody(i, _):
            acc = part_v[0, pl.ds(i * L, L)]
            for p in range(1, NS):
                acc = acc + part_v[p, pl.ds(i * L, L)]
            red_v[pl.ds(i * L, L)] = acc
            return 0

        lax.fori_loop(0, npad_per_tile // L, rbody, 0)
        pltpu.sync_copy(
            red_v, deg_hbm.at[pl.ds(c * n_pad + s * npad_per_tile, npad_per_tile)]
        )

    return deg_kernel


@functools.lru_cache(maxsize=None)
def _agg_call(e_rows: int, n_pad: int):
    cpt = e_rows // NS
    npad_per_tile = n_pad // NS

    @functools.partial(
        pl.kernel,
        out_type=jax.ShapeDtypeStruct((NC, n_pad, FH), jnp.float32),
        mesh=_sc_mesh(),
        compiler_params=pltpu.CompilerParams(needs_layout_passes=False),
        scratch_types=[
            pltpu.VMEM((cpt // WN, CW), jnp.int32),
            pltpu.VMEM((cpt // WN, CW), jnp.int32),
            pltpu.VMEM((NB, CW, FH), jnp.float32),
            pltpu.VMEM_SHARED((n_pad, FH), jnp.float32),
            [pltpu.SemaphoreType.DMA] * NB,
            [pltpu.SemaphoreType.DMA] * NB,
        ],
    )
    def agg_kernel(h_hbm, src_hbm, dst_hbm, out_hbm, src_v, dst_v, rows_v, acc_sh, gs, ss):
        c = lax.axis_index("c")
        s = lax.axis_index("s")
        hb = cpt // WN

        def zbody(i, _):
            r = i // (FH // L)
            j = i % (FH // L)
            rows_v[0, r, pl.ds(j * L, L)] = jnp.zeros((L,), jnp.float32)
            return 0

        lax.fori_loop(0, CW * FH // L, zbody, 0)
        for t in range(npad_per_tile // CW):
            pltpu.sync_copy(
                rows_v.at[0], acc_sh.at[pl.ds(s * npad_per_tile + t * CW, CW)]
            )
        plsc.subcore_barrier()

        for h in range(WN):
            pltpu.sync_copy(
                src_hbm.at[pl.ds(c * e_rows + s * cpt + h * hb, hb)], src_v
            )
            pltpu.sync_copy(dst_hbm.at[pl.ds(s * cpt + h * hb, hb)], dst_v)
            for b in range(NB - 1):
                pltpu.async_copy(h_hbm.at[src_v.at[b]], rows_v.at[b], gs[b])

            @pl.loop(0, hb, step=NB)
            def ebody(t):
                for b in range(NB):
                    chunk = t + b
                    nxt = chunk + NB - 1
                    nb = (b + NB - 1) % NB

                    pltpu.make_async_copy(
                        h_hbm.at[src_v.at[chunk]], rows_v.at[b], gs[b]
                    ).wait()
                    pltpu.async_copy(
                        rows_v.at[b], acc_sh.at[dst_v.at[chunk]], ss[b], add=True
                    )

                    @pl.when(nxt < hb)
                    def _():
                        @pl.when(chunk >= 1)
                        def _():
                            pltpu.make_async_copy(
                                rows_v.at[nb],
                                acc_sh.at[dst_v.at[chunk - 1]],
                                ss[nb],
                            ).wait()

                        pltpu.async_copy(
                            h_hbm.at[src_v.at[nxt]], rows_v.at[nb], gs[nb]
                        )

            for d in range(NB):
                cd = hb - NB + d
                pltpu.make_async_copy(
                    rows_v.at[cd % NB], acc_sh.at[dst_v.at[cd]], ss[cd % NB]
                ).wait()

        plsc.subcore_barrier()
        pltpu.sync_copy(
            acc_sh.at[pl.ds(s * npad_per_tile, npad_per_tile)],
            out_hbm.at[c, pl.ds(s * npad_per_tile, npad_per_tile)],
        )

    return agg_kernel


def _scale_call(x_pad, deg_cols, n_pad):
    rb = 1024

    def body(x_ref, deg_ref, h_ref, norms_ref):
        ns = lax.rsqrt(jnp.clip(deg_ref[:, 0:1], 1.0, None))
        nd = lax.rsqrt(jnp.clip(deg_ref[:, 1:2], 1.0, None))
        norms_ref[:, 0:1] = ns
        norms_ref[:, 1:2] = nd
        h = x_ref[:] * ns
        h_ref[0] = h[:, :FH]
        h_ref[1] = h[:, FH:]

    return pl.pallas_call(
        body,
        grid=(n_pad // rb,),
        in_specs=[
            pl.BlockSpec((rb, F), lambda i: (i, 0)),
            pl.BlockSpec((rb, 2), lambda i: (i, 0)),
        ],
        out_specs=[
            pl.BlockSpec((NC, rb, FH), lambda i: (0, i, 0)),
            pl.BlockSpec((rb, 2), lambda i: (i, 0)),
        ],
        out_shape=[
            jax.ShapeDtypeStruct((NC, n_pad, FH), jnp.float32),
            jax.ShapeDtypeStruct((n_pad, 2), jnp.float32),
        ],
    )(x_pad, deg_cols)


def _mid_call(agg1, norms, W1, b1, W2, n_pad):
    rb = 512
    h1f = W1.shape[1]

    def body(agg_ref, norms_ref, w1_ref, b1_ref, w2_ref, out_ref):
        a = jnp.concatenate([agg_ref[0], agg_ref[1]], axis=1)
        h = a * norms_ref[:, 1:2]
        h1 = jnp.dot(h, w1_ref[:], preferred_element_type=jnp.float32) + b1_ref[:]
        h1 = jnp.where(h1 > 0, h1, 0.01 * h1)
        m = jnp.dot(h1, w2_ref[:], preferred_element_type=jnp.float32)
        ms = m * norms_ref[:, 0:1]
        out_ref[0] = ms[:, :FH]
        out_ref[1] = ms[:, FH:]

    return pl.pallas_call(
        body,
        grid=(n_pad // rb,),
        in_specs=[
            pl.BlockSpec((NC, rb, FH), lambda i: (0, i, 0)),
            pl.BlockSpec((rb, 2), lambda i: (i, 0)),
            pl.BlockSpec((F, h1f), lambda i: (0, 0)),
            pl.BlockSpec((1, h1f), lambda i: (0, 0)),
            pl.BlockSpec((h1f, F), lambda i: (0, 0)),
        ],
        out_specs=pl.BlockSpec((NC, rb, FH), lambda i: (0, i, 0)),
        out_shape=jax.ShapeDtypeStruct((NC, n_pad, FH), jnp.float32),
    )(agg1, norms, W1, b1, W2)


def _readout_call(agg2, norms, b2, batch):
    def body(agg_ref, norms_ref, b2_ref, out_ref):
        a = jnp.concatenate([agg_ref[0], agg_ref[1]], axis=1)
        h2 = a * norms_ref[:, 1:2] + b2_ref[:]
        h2 = jnp.where(h2 > 0, h2, 0.01 * h2)
        out_ref[0] = jnp.mean(h2, axis=0, keepdims=True)

    out = pl.pallas_call(
        body,
        grid=(batch,),
        in_specs=[
            pl.BlockSpec((NC, GRAPH, FH), lambda i: (0, i, 0)),
            pl.BlockSpec((GRAPH, 2), lambda i: (i, 0)),
            pl.BlockSpec((1, F), lambda i: (0, 0)),
        ],
        out_specs=pl.BlockSpec((1, 1, F), lambda i: (i, 0, 0)),
        out_shape=jax.ShapeDtypeStruct((batch, 1, F), jnp.float32),
    )(agg2, norms, b2)
    return out.reshape(batch, F)


def kernel(x, edge_index, W1, b1, W2, b2):
    n = x.shape[0]
    batch = n // GRAPH
    n_pad = ((n + 1 + 1023) // 1024) * 1024

    src = edge_index[0].astype(jnp.int32)
    dst = edge_index[1].astype(jnp.int32)
    e = src.shape[0]
    ec = NS * K * 8
    e_pad = ((e + ec - 1) // ec) * ec
    e_rows = e_pad // CW
    src_p = jnp.full((e_pad,), n, jnp.int32).at[:e].set(src)
    dst_p = jnp.full((e_pad,), n, jnp.int32).at[:e].set(dst)
    idx2 = jnp.concatenate([src_p, dst_p])
    src2 = jnp.concatenate([src_p, src_p + n_pad]).reshape(NC * e_rows, CW)
    dst2 = dst_p.reshape(e_rows, CW)

    x_pad = jnp.zeros((n_pad, F), jnp.float32).at[:n].set(x)

    deg = _degree_call(e_pad, n_pad)(idx2)
    h_split, norms = _scale_call(x_pad, deg.reshape(NC, n_pad).T, n_pad)
    agg1 = _agg_call(e_rows, n_pad)(
        h_split.reshape(NC * n_pad, FH), src2, dst2
    )
    m_split = _mid_call(agg1, norms, W1, b1.reshape(1, -1), W2, n_pad)
    agg2 = _agg_call(e_rows, n_pad)(
        m_split.reshape(NC * n_pad, FH), src2, dst2
    )
    return _readout_call(agg2, norms, b2.reshape(1, -1), batch)
